# baseline (speedup 1.0000x reference)
import functools

import jax
import jax.numpy as jnp
from jax import lax
from jax.experimental import pallas as pl
from jax.experimental.pallas import tpu as pltpu
from jax.experimental.pallas import tpu_sc as plsc

B = 16384
D_ITEM = 64
D_SPARSE = 32
MM_DIM = 128
D_MM = 32
DNN_HID = 256
HID_OUT = 128

_NC = 2
_NS = 16
_NW = _NC * _NS
_BPW = B // _NW
_CHUNK = 128
_NCHUNK = _BPW // _CHUNK

_BLK = 1024
_GRID = B // _BLK


def _sc_gather_body(seq2_ref, cate_ref, brand_ref, shop_ref, tag_ref,
                    t_item2, t_sp4,
                    o_item, o_cate, o_brand, o_shop, o_tag,
                    idx_it, idx_sp,
                    r_item, r_sp,
                    s_item, s_sp):
    wid = lax.axis_index("s") * _NC + lax.axis_index("c")
    base = wid * _BPW
    in_refs = (seq2_ref, cate_ref, brand_ref, shop_ref, tag_ref)
    outs = (o_item, o_cate, o_brand, o_shop, o_tag)
    half = _BPW // 2

    pltpu.sync_copy(in_refs[0].at[pl.ds(base, _BPW)], idx_it)
    item_handles = []
    for j in range(_NCHUNK):
        item_handles.append(pltpu.async_copy(
            t_item2.at[idx_it.at[pl.ds(j * _CHUNK, _CHUNK)]],
            r_item.at[pl.ds(j * _CHUNK, _CHUNK)],
            s_item))

    for i in range(1, 5):
        pltpu.sync_copy(in_refs[i].at[pl.ds(base, _BPW)], idx_sp)
        for w in range(2):
            hs = []
            for j in range(_NCHUNK // 2):
                c = w * (_NCHUNK // 2) + j
                hs.append(pltpu.async_copy(
                    t_sp4.at[idx_sp.at[pl.ds(c * _CHUNK, _CHUNK)]],
                    r_sp.at[pl.ds(j * _CHUNK, _CHUNK)],
                    s_sp))
            for h in hs:
                h.wait()
            pltpu.sync_copy(r_sp, outs[i].at[pl.ds(base + w * half, half)])

    for h in item_handles:
        h.wait()
    pltpu.sync_copy(r_item, o_item.at[pl.ds(base, _BPW)])


@jax.jit
def _sc_gather(seq2, cate_id, brand_id, shop_id, tag_id, t_item2, t_sp4):
    mesh = plsc.VectorSubcoreMesh(core_axis_name="c", subcore_axis_name="s")
    f32 = jnp.float32
    out_type = [jax.ShapeDtypeStruct((B, 128), f32) for _ in range(5)]
    scratch = (
        [pltpu.VMEM((_BPW,), jnp.int32), pltpu.VMEM((_BPW,), jnp.int32)]
        + [pltpu.VMEM((_BPW, 128), f32), pltpu.VMEM((_BPW // 2, 128), f32)]
        + [pltpu.SemaphoreType.DMA, pltpu.SemaphoreType.DMA]
    )
    return pl.kernel(
        _sc_gather_body,
        out_type=out_type,
        mesh=mesh,
        scratch_types=scratch,
        compiler_params=pltpu.CompilerParams(use_tc_tiling_on_sc=True),
    )(seq2, cate_id, brand_id, shop_id, tag_id, t_item2, t_sp4)


def _mlp_body(gi2, par, gc, gb, gs, gt, dns, mm,
              mmW, mmb, w1i, w1c, w1b, w1s, w1t, w1d, w1m, b1, w2, b2,
              out):
    f32 = jnp.float32
    gi = jnp.where(par[...] > 0, gi2[..., D_ITEM:], gi2[..., :D_ITEM])
    acc = jnp.dot(gi, w1i[...], preferred_element_type=f32)
    acc += jnp.dot(gc[..., 0:32], w1c[...], preferred_element_type=f32)
    acc += jnp.dot(gb[..., 32:64], w1b[...], preferred_element_type=f32)
    acc += jnp.dot(gs[..., 64:96], w1s[...], preferred_element_type=f32)
    acc += jnp.dot(gt[..., 96:128], w1t[...], preferred_element_type=f32)
    acc += jnp.dot(dns[...], w1d[...], preferred_element_type=f32)
    mmp = jnp.dot(mm[...], mmW[...], preferred_element_type=f32) + mmb[...]
    acc += jnp.dot(mmp, w1m[...], preferred_element_type=f32)
    acc += b1[...]
    h = jnp.maximum(acc, 0.0)
    out[...] = jnp.dot(h, w2[...], preferred_element_type=f32) + b2[...]


def _full(shape):
    return pl.BlockSpec(shape, lambda i: (0, 0))


def _mlp(gi2, par, gc, gb, gs, gt, dns, mm, mmW, mmb,
         w1i, w1c, w1b, w1s, w1t, w1d, w1m, b1, w2, b2):
    blk = lambda d: pl.BlockSpec((_BLK, d), lambda i: (i, 0))
    in_specs = [
        blk(128), blk(1),
        blk(128), blk(128), blk(128), blk(128),
        blk(3), blk(MM_DIM),
        _full((MM_DIM, D_MM)), _full((1, D_MM)),
        _full((D_ITEM, DNN_HID)),
        _full((D_SPARSE, DNN_HID)), _full((D_SPARSE, DNN_HID)),
        _full((D_SPARSE, DNN_HID)), _full((D_SPARSE, DNN_HID)),
        _full((3, DNN_HID)), _full((D_MM, DNN_HID)),
        _full((1, DNN_HID)),
        _full((DNN_HID, HID_OUT)), _full((1, HID_OUT)),
    ]
    return pl.pallas_call(
        _mlp_body,
        grid=(_GRID,),
        in_specs=in_specs,
        out_specs=pl.BlockSpec((_BLK, HID_OUT), lambda i: (i, 0)),
        out_shape=jax.ShapeDtypeStruct((B, HID_OUT), jnp.float32),
        compiler_params=pltpu.CompilerParams(
            dimension_semantics=("arbitrary",)),
    )(gi2, par, gc, gb, gs, gt, dns, mm, mmW, mmb,
      w1i, w1c, w1b, w1s, w1t, w1d, w1m, b1, w2, b2)


def kernel(seq_id, cate_id, brand_id, shop_id, tag_id,
           dense_0, dense_1, dense_2, mm_emb_0,
           emb_item, emb_cate, emb_brand, emb_shop, emb_tag,
           mm_W, mm_b, W1, b1, W2, b2):
    i32 = jnp.int32
    seq = seq_id.astype(i32)
    t_item2 = emb_item[:1000000].reshape(500000, 128)
    t_sp4 = jnp.concatenate(
        [emb_cate[:100000], emb_brand[:100000],
         emb_shop[:100000], emb_tag[:100000]], axis=1)
    gi2, gc, gb, gs, gt = _sc_gather(
        seq >> 1, cate_id.astype(i32), brand_id.astype(i32),
        shop_id.astype(i32), tag_id.astype(i32),
        t_item2, t_sp4)
    par = (seq & 1).astype(jnp.float32).reshape(B, 1)

    dns = jnp.stack([dense_0, dense_1, dense_2], axis=1)
    w1i = W1[:D_ITEM]
    o = D_ITEM
    w1c = W1[o:o + D_SPARSE]; o += D_SPARSE
    w1b = W1[o:o + D_SPARSE]; o += D_SPARSE
    w1s = W1[o:o + D_SPARSE]; o += D_SPARSE
    w1t = W1[o:o + D_SPARSE]; o += D_SPARSE
    w1d = W1[o:o + 3]; o += 3
    w1m = W1[o:o + D_MM]

    return _mlp(gi2, par, gc, gb, gs, gt, dns, mm_emb_0,
                mm_W, mm_b.reshape(1, -1),
                w1i, w1c, w1b, w1s, w1t, w1d, w1m,
                b1.reshape(1, -1), W2, b2.reshape(1, -1))
